# Initial kernel scaffold; baseline (speedup 1.0000x reference)
import jax
import jax.numpy as jnp
from jax.experimental import pallas as pl


def kernel(x, emb_table, weight, bias, logits):
    raise NotImplementedError("write your pallas kernel here")


# speedup vs baseline: 75.6741x; 75.6741x over previous
import math

import jax
import jax.numpy as jnp
import numpy as np
from jax.experimental import pallas as pl
from jax.experimental.pallas import tpu as pltpu

NUM_NODES = 160
SEQ_LEN = 128
BATCH = 8
TOPK = 20

_DINV = np.float32(np.float32(20.0) ** np.float32(-0.5))
_NORM = np.float32(_DINV * _DINV)
_NEG = np.float32(-3.0e38)


def _adj_kernel(emb_ref, l0_ref, l1_ref, g0_ref, g1_ref, a_ref):
    emb = emb_ref[...]
    dot = jax.lax.dot_general(
        emb, emb, (((1,), (1,)), ((), ())), preferred_element_type=jnp.float32
    )
    row_i = jax.lax.broadcasted_iota(jnp.int32, (NUM_NODES, NUM_NODES), 0)
    col_i = jax.lax.broadcasted_iota(jnp.int32, (NUM_NODES, NUM_NODES), 1)
    eye = (row_i == col_i).astype(jnp.float32)
    n2_col = jnp.sum(emb * emb, axis=1, keepdims=True)
    n2_row = jnp.max(eye * n2_col, axis=0, keepdims=True)
    cos = dot / (jnp.sqrt(n2_col) * jnp.sqrt(n2_row))

    gate = (l0_ref[...] + g0_ref[...] >= l1_ref[...] + g1_ref[...]).astype(
        jnp.float32
    )

    acc = jnp.zeros((BATCH, NUM_NODES, NUM_NODES), jnp.float32)
    cosm = cos
    for t in range(TOPK):
        mx = jnp.max(cosm, axis=1, keepdims=True)
        jstar = jnp.min(
            jnp.where(cosm >= mx, col_i, np.int32(NUM_NODES)), axis=1, keepdims=True
        )
        m = (col_i == jstar).astype(jnp.float32)
        acc = acc + gate[:, :, t][:, :, None] * m[None, :, :]
        cosm = jnp.where(m > 0.0, _NEG, cosm)
    a_ref[...] = acc * _NORM


def _mm_kernel(x_ref, w_ref, a_ref, b_ref, o_ref):
    xb = x_ref[0]
    w = w_ref[...]
    h = jax.lax.dot_general(
        w, xb, (((0,), (0,)), ((), ())), preferred_element_type=jnp.float32
    )
    bias = b_ref[...]
    for bn in range(BATCH):
        hb = h[:, bn * NUM_NODES : (bn + 1) * NUM_NODES]
        ob = jax.lax.dot_general(
            hb, a_ref[bn], (((1,), (1,)), ((), ())), preferred_element_type=jnp.float32
        )
        o_ref[0, :, bn * NUM_NODES : (bn + 1) * NUM_NODES] = ob + bias


def kernel(x, emb_table, weight, bias, logits):
    n_total = BATCH * NUM_NODES
    g = jax.random.gumbel(jax.random.key(42), logits.shape, logits.dtype)
    l0 = logits[:, 0].reshape(BATCH, NUM_NODES, TOPK)
    l1 = logits[:, 1].reshape(BATCH, NUM_NODES, TOPK)
    g0 = g[:, 0].reshape(BATCH, NUM_NODES, TOPK)
    g1 = g[:, 1].reshape(BATCH, NUM_NODES, TOPK)

    a = pl.pallas_call(
        _adj_kernel,
        out_shape=jax.ShapeDtypeStruct((BATCH, NUM_NODES, NUM_NODES), jnp.float32),
    )(emb_table, l0, l1, g0, g1)

    out = pl.pallas_call(
        _mm_kernel,
        grid=(BATCH,),
        in_specs=[
            pl.BlockSpec((1, SEQ_LEN, n_total), lambda bo: (bo, 0, 0)),
            pl.BlockSpec((SEQ_LEN, SEQ_LEN), lambda bo: (0, 0)),
            pl.BlockSpec((BATCH, NUM_NODES, NUM_NODES), lambda bo: (0, 0, 0)),
            pl.BlockSpec((SEQ_LEN, 1), lambda bo: (0, 0)),
        ],
        out_specs=pl.BlockSpec((1, SEQ_LEN, n_total), lambda bo: (bo, 0, 0)),
        out_shape=jax.ShapeDtypeStruct((BATCH, SEQ_LEN, n_total), jnp.float32),
    )(x, weight, a, bias.reshape(SEQ_LEN, 1))
    return out


# speedup vs baseline: 81.0283x; 1.0708x over previous
import math

import jax
import jax.numpy as jnp
import numpy as np
from jax.experimental import pallas as pl
from jax.experimental.pallas import tpu as pltpu

NUM_NODES = 160
SEQ_LEN = 128
BATCH = 8
TOPK = 20

_DINV = np.float32(np.float32(20.0) ** np.float32(-0.5))
_NORM = np.float32(_DINV * _DINV)
_NEG = np.float32(-3.0e38)


def _fused_kernel(emb_ref, l0_ref, l1_ref, g0_ref, g1_ref, x_ref, w_ref, b_ref,
                  o_ref, a_scr):
    i = pl.program_id(0)

    @pl.when(i == 0)
    def _build_adjacency():
        emb = emb_ref[...]
        dot = jax.lax.dot_general(
            emb, emb, (((1,), (1,)), ((), ())), preferred_element_type=jnp.float32
        )
        row_i = jax.lax.broadcasted_iota(jnp.int32, (NUM_NODES, NUM_NODES), 0)
        col_i = jax.lax.broadcasted_iota(jnp.int32, (NUM_NODES, NUM_NODES), 1)
        eye = (row_i == col_i).astype(jnp.float32)
        n2_col = jnp.sum(emb * emb, axis=1, keepdims=True)
        n2_row = jnp.max(eye * n2_col, axis=0, keepdims=True)
        cos = dot / (jnp.sqrt(n2_col) * jnp.sqrt(n2_row))

        gate = (l0_ref[...] + g0_ref[...] >= l1_ref[...] + g1_ref[...]).astype(
            jnp.float32
        )

        acc = jnp.zeros((BATCH, NUM_NODES, NUM_NODES), jnp.float32)
        cosm = cos
        for t in range(TOPK):
            mx = jnp.max(cosm, axis=1, keepdims=True)
            jstar = jnp.min(
                jnp.where(cosm >= mx, col_i, np.int32(NUM_NODES)),
                axis=1, keepdims=True,
            )
            m = (col_i == jstar).astype(jnp.float32)
            acc = acc + gate[:, :, t][:, :, None] * m[None, :, :]
            cosm = jnp.where(m > 0.0, _NEG, cosm)
        a_scr[...] = acc * _NORM

    xb = x_ref[0]
    w = w_ref[...]
    h = jax.lax.dot_general(
        w, xb, (((0,), (0,)), ((), ())), preferred_element_type=jnp.float32
    )
    bias = b_ref[...]
    for bn in range(BATCH):
        hb = h[:, bn * NUM_NODES : (bn + 1) * NUM_NODES]
        ob = jax.lax.dot_general(
            hb, a_scr[bn], (((1,), (1,)), ((), ())),
            preferred_element_type=jnp.float32,
        )
        o_ref[0, :, bn * NUM_NODES : (bn + 1) * NUM_NODES] = ob + bias


def kernel(x, emb_table, weight, bias, logits):
    n_total = BATCH * NUM_NODES
    g = jax.random.gumbel(jax.random.key(42), logits.shape, logits.dtype)
    l0 = logits[:, 0].reshape(BATCH, NUM_NODES, TOPK)
    l1 = logits[:, 1].reshape(BATCH, NUM_NODES, TOPK)
    g0 = g[:, 0].reshape(BATCH, NUM_NODES, TOPK)
    g1 = g[:, 1].reshape(BATCH, NUM_NODES, TOPK)

    zero3 = lambda i: (0, 0, 0)
    out = pl.pallas_call(
        _fused_kernel,
        grid=(BATCH,),
        in_specs=[
            pl.BlockSpec((NUM_NODES, 64), lambda i: (0, 0)),
            pl.BlockSpec((BATCH, NUM_NODES, TOPK), zero3),
            pl.BlockSpec((BATCH, NUM_NODES, TOPK), zero3),
            pl.BlockSpec((BATCH, NUM_NODES, TOPK), zero3),
            pl.BlockSpec((BATCH, NUM_NODES, TOPK), zero3),
            pl.BlockSpec((1, SEQ_LEN, n_total), lambda i: (i, 0, 0)),
            pl.BlockSpec((SEQ_LEN, SEQ_LEN), lambda i: (0, 0)),
            pl.BlockSpec((SEQ_LEN, 1), lambda i: (0, 0)),
        ],
        out_specs=pl.BlockSpec((1, SEQ_LEN, n_total), lambda i: (i, 0, 0)),
        out_shape=jax.ShapeDtypeStruct((BATCH, SEQ_LEN, n_total), jnp.float32),
        scratch_shapes=[pltpu.VMEM((BATCH, NUM_NODES, NUM_NODES), jnp.float32)],
    )(emb_table, l0, l1, g0, g1, x, weight, bias.reshape(SEQ_LEN, 1))
    return out
